# baseline (speedup 1.0000x reference)
import jax
import jax.numpy as jnp
from jax import lax
from jax.experimental import pallas as pl
from jax.experimental.pallas import tpu as pltpu
from jax.experimental.pallas import tpu_sc as plsc

N = 1_600_000
B = 100_000
Np = 1_638_400
Bp = 102_400
ROWS2D = Np // 128
NW = 32
RPT = Np // NW
KCH = 6400
NCH = RPT // KCH
WIN = Bp // NW
Npp = Np + KCH

_SC_PARAMS = pltpu.CompilerParams(use_tc_tiling_on_sc=False,
                                  needs_layout_passes=False)


def _gather_body(tab_hbm, batch_hbm, pos_hbm, out_hbm,
                 idx_v, rows_v, pos_v, pln_v, sem):
    cid = lax.axis_index("c")
    sid = lax.axis_index("s")
    wid = sid * 2 + cid
    lane = lax.iota(jnp.int32, 16)
    lane3 = lane * 3
    for k in range(NCH):
        base = wid * RPT + k * KCH
        pltpu.sync_copy(batch_hbm.at[pl.ds(base, KCH)], idx_v)
        gcp = pltpu.async_copy(tab_hbm.at[idx_v], rows_v, sem)
        pltpu.sync_copy(pos_hbm.at[pl.ds(base * 3, KCH * 3)], pos_v)
        gcp.wait()

        def grp(i, _):
            r16 = lane + i * 16
            for c in range(4):
                v = plsc.load_gather(rows_v, [r16, jnp.full((16,), c, jnp.int32)])
                pln_v[c, pl.ds(i * 16, 16)] = v
            for c in range(3):
                v = plsc.load_gather(pos_v, [lane3 + (i * 48 + c)])
                pln_v[4 + c, pl.ds(i * 16, 16)] = v
            return 0

        lax.fori_loop(0, KCH // 16, grp, 0)
        for j in range(7):
            pltpu.sync_copy(pln_v.at[j], out_hbm.at[j, pl.ds(base, KCH)])


_gather_call = pl.kernel(
    _gather_body,
    out_type=jax.ShapeDtypeStruct((7, Np), jnp.float32),
    mesh=plsc.VectorSubcoreMesh(core_axis_name="c", subcore_axis_name="s"),
    compiler_params=_SC_PARAMS,
    scratch_types=[
        pltpu.VMEM((KCH,), jnp.int32),
        pltpu.VMEM((KCH, 8), jnp.float32),
        pltpu.VMEM((KCH * 3,), jnp.float32),
        pltpu.VMEM((7, KCH), jnp.float32),
        pltpu.SemaphoreType.DMA,
    ],
)


def _mlp_body(t_ref, g_ref, W0, b0, W1, b1, W2, b2, W3, b3, W4, b4,
              cx_ref, cy_ref, cz_ref):
    tb = t_ref[...]
    gt, gx, gy, gz = g_ref[0], g_ref[1], g_ref[2], g_ref[3]
    px, py, pz = g_ref[4], g_ref[5], g_ref[6]
    s = jnp.sign(tb - gt)
    dx = px - gx
    dy = py - gy
    dz = pz - gz
    r2 = dx * dx + dy * dy + dz * dz

    h = [s, r2]
    for W, b, fin, fout, relu in (
        (W0, b0, 2, 10, True),
        (W1, b1, 10, 20, True),
        (W2, b2, 20, 10, True),
        (W3, b3, 10, 5, True),
        (W4, b4, 5, 1, False),
    ):
        nxt = []
        for j in range(fout):
            acc = h[0] * W[j, 0] + b[j]
            for k in range(1, fin):
                acc = acc + h[k] * W[j, k]
            nxt.append(jnp.maximum(acc, 0.0) if relu else acc)
        h = nxt

    f = h[0] * lax.rsqrt(jnp.maximum(r2, 1e-24))
    cx_ref[...] = f * dx
    cy_ref[...] = f * dy
    cz_ref[...] = f * dz


def _mlp_call(t2, g7, *wb):
    BR = 512
    grid = (ROWS2D // BR,)
    smem = pl.BlockSpec(memory_space=pltpu.MemorySpace.SMEM)
    return pl.pallas_call(
        _mlp_body,
        grid=grid,
        in_specs=[
            pl.BlockSpec((BR, 128), lambda i: (i, 0)),
            pl.BlockSpec((7, BR, 128), lambda i: (0, i, 0)),
        ] + [smem] * 10,
        out_specs=[pl.BlockSpec((BR, 128), lambda i: (i, 0))] * 3,
        out_shape=[jax.ShapeDtypeStruct((ROWS2D + 50, 128), jnp.float32)] * 3,
    )(t2, g7, *wb)


def _scatter_body(cx_hbm, cy_hbm, cz_hbm, b_hbm, bounds_hbm, out_hbm,
                  bounds_v, idx_v, c_v, acc):
    cid = lax.axis_index("c")
    sid = lax.axis_index("s")
    wid = sid * 2 + cid
    pltpu.sync_copy(bounds_hbm, bounds_v)
    bv = bounds_v[pl.ds(wid, 16)]
    r_lo = bv[0]
    r_hi = bv[1]
    start = (r_lo // 8) * 8
    nch = (r_hi - start + KCH - 1) // KCH
    wbase = wid * WIN

    def zero(i, _):
        acc[pl.ds(i * 16, 16)] = jnp.zeros((16,), jnp.float32)
        return 0

    lax.fori_loop(0, WIN * 4 // 16, zero, 0)

    lane = lax.iota(jnp.int32, 16)

    def chunk(k, _):
        off = start + k * KCH
        pltpu.sync_copy(b_hbm.at[pl.ds(off, KCH)], idx_v)
        pltpu.sync_copy(cx_hbm.at[pl.ds(off, KCH)], c_v.at[0])
        pltpu.sync_copy(cy_hbm.at[pl.ds(off, KCH)], c_v.at[1])
        pltpu.sync_copy(cz_hbm.at[pl.ds(off, KCH)], c_v.at[2])

        def grp(i, _):
            ids = idx_v[pl.ds(i * 16, 16)]
            rowpos = lane + (i * 16 + off)
            valid = (rowpos >= r_lo) & (rowpos < r_hi)
            local = jnp.where(valid, ids - wbase, 0)
            flat = local * 4
            for c in range(3):
                vals = c_v[c, pl.ds(i * 16, 16)]
                plsc.addupdate_scatter(acc, [flat + c], vals, mask=valid)
            return 0

        lax.fori_loop(0, KCH // 16, grp, 0)
        return 0

    lax.fori_loop(0, nch, chunk, 0)
    pltpu.sync_copy(acc, out_hbm.at[pl.ds(wid * WIN * 4, WIN * 4)])


_scatter_call = pl.kernel(
    _scatter_body,
    out_type=jax.ShapeDtypeStruct((Bp * 4,), jnp.float32),
    mesh=plsc.VectorSubcoreMesh(core_axis_name="c", subcore_axis_name="s"),
    compiler_params=_SC_PARAMS,
    scratch_types=[
        pltpu.VMEM((48,), jnp.int32),
        pltpu.VMEM((KCH,), jnp.int32),
        pltpu.VMEM((3, KCH), jnp.float32),
        pltpu.VMEM((WIN * 4,), jnp.float32),
    ],
)


def _norm_body(a_ref, o_ref):
    p = a_ref[...]
    sq = p * p
    r = lax.broadcasted_iota(jnp.int32, (128, 128), 0)
    c = lax.broadcasted_iota(jnp.int32, (128, 128), 1)
    M = ((r // 4) == (c // 4)).astype(jnp.float32)
    n2 = lax.dot_general(sq, M, (((1,), (0,)), ((), ())),
                         preferred_element_type=jnp.float32)
    n = jnp.sqrt(jnp.maximum(n2, 1e-24))
    o_ref[...] = p / n


def _norm_call(a):
    BRn = 400
    rows = (Bp * 4) // 128
    return pl.pallas_call(
        _norm_body,
        grid=(rows // BRn,),
        in_specs=[pl.BlockSpec((BRn, 128), lambda i: (i, 0))],
        out_specs=pl.BlockSpec((BRn, 128), lambda i: (i, 0)),
        out_shape=jax.ShapeDtypeStruct((rows, 128), jnp.float32),
    )(a)


def kernel(t, pos, poi_t, poi_pos, batch, W0, b0, W1, b1, W2, b2, W3, b3, W4, b4):
    f32 = jnp.float32
    tab = jnp.concatenate(
        [poi_t[:, None], poi_pos, jnp.zeros((B, 4), f32)], axis=1)
    tab = jnp.pad(tab, ((0, Bp - B), (0, 0)))

    batch_pp = jnp.concatenate(
        [batch, jnp.full((Npp - N,), Bp - 1, jnp.int32)])
    batch_p = batch_pp[:Np]
    pos_flat = jnp.pad(pos, ((0, Np - N), (0, 0))).reshape(Np * 3)

    g7 = _gather_call(tab, batch_p, pos_flat)

    t2 = jnp.pad(t, (0, Np - N)).reshape(ROWS2D, 128)
    g7r = g7.reshape(7, ROWS2D, 128)

    cx, cy, cz = _mlp_call(t2, g7r,
                           W0, b0, W1, b1, W2, b2, W3, b3, W4, b4)

    bounds = jnp.searchsorted(
        batch_p, jnp.arange(33, dtype=jnp.int32) * WIN).astype(jnp.int32)
    bounds = jnp.pad(bounds, (0, 15))

    acc = _scatter_call(cx.reshape(Npp), cy.reshape(Npp), cz.reshape(Npp),
                        batch_pp, bounds)

    o = _norm_call(acc.reshape((Bp * 4) // 128, 128))
    return o.reshape(Bp, 4)[:B, :3]


# speedup vs baseline: 6.4402x; 6.4402x over previous
import jax
import jax.numpy as jnp
from jax import lax
from jax.experimental import pallas as pl
from jax.experimental.pallas import tpu as pltpu
from jax.experimental.pallas import tpu_sc as plsc

N = 1_600_000
B = 100_000
Np = 1_638_400
Bp = 102_400
ROWS2D = Np // 128
NW = 32
RPT = Np // NW
KCH = 6400
NCH = RPT // KCH
WIN = Bp // NW
Npp = Np + KCH

_SC_PARAMS = pltpu.CompilerParams(use_tc_tiling_on_sc=False,
                                  needs_layout_passes=False)


def _gather_body(tab_hbm, batch_hbm, out_hbm, idx_v, rows_v, pln_v, sem):
    cid = lax.axis_index("c")
    sid = lax.axis_index("s")
    wid = sid * 2 + cid
    lane = lax.iota(jnp.int32, 16)
    for k in range(NCH):
        base = wid * RPT + k * KCH
        pltpu.sync_copy(batch_hbm.at[pl.ds(base, KCH)], idx_v)
        pltpu.async_copy(tab_hbm.at[idx_v], rows_v, sem).wait()

        def grp(i, _):
            r16 = lane + i * 16
            for c in range(4):
                v = plsc.load_gather(rows_v, [r16, jnp.full((16,), c, jnp.int32)])
                pln_v[c, pl.ds(i * 16, 16)] = v
            return 0

        lax.fori_loop(0, KCH // 16, grp, 0)
        for j in range(4):
            pltpu.sync_copy(pln_v.at[j], out_hbm.at[j, pl.ds(base, KCH)])


_gather_call = pl.kernel(
    _gather_body,
    out_type=jax.ShapeDtypeStruct((4, Np), jnp.float32),
    mesh=plsc.VectorSubcoreMesh(core_axis_name="c", subcore_axis_name="s"),
    compiler_params=_SC_PARAMS,
    scratch_types=[
        pltpu.VMEM((KCH,), jnp.int32),
        pltpu.VMEM((KCH, 8), jnp.float32),
        pltpu.VMEM((4, KCH), jnp.float32),
        pltpu.SemaphoreType.DMA,
    ],
)


def _mlp_body(t_ref, g_ref, px_ref, py_ref, pz_ref,
              W0, b0, W1, b1, W2, b2, W3, b3, W4, b4,
              cx_ref, cy_ref, cz_ref):
    tb = t_ref[...]
    gt, gx, gy, gz = g_ref[0], g_ref[1], g_ref[2], g_ref[3]
    px, py, pz = px_ref[...], py_ref[...], pz_ref[...]
    s = jnp.sign(tb - gt)
    dx = px - gx
    dy = py - gy
    dz = pz - gz
    r2 = dx * dx + dy * dy + dz * dz

    h = [s, r2]
    for W, b, fin, fout, relu in (
        (W0, b0, 2, 10, True),
        (W1, b1, 10, 20, True),
        (W2, b2, 20, 10, True),
        (W3, b3, 10, 5, True),
        (W4, b4, 5, 1, False),
    ):
        nxt = []
        for j in range(fout):
            acc = h[0] * W[j, 0] + b[j]
            for k in range(1, fin):
                acc = acc + h[k] * W[j, k]
            nxt.append(jnp.maximum(acc, 0.0) if relu else acc)
        h = nxt

    f = h[0] * lax.rsqrt(jnp.maximum(r2, 1e-24))
    cx_ref[...] = f * dx
    cy_ref[...] = f * dy
    cz_ref[...] = f * dz


def _mlp_call(t2, g4, px, py, pz, *wb):
    BR = 512
    grid = (ROWS2D // BR,)
    smem = pl.BlockSpec(memory_space=pltpu.MemorySpace.SMEM)
    row_spec = pl.BlockSpec((BR, 128), lambda i: (i, 0))
    return pl.pallas_call(
        _mlp_body,
        grid=grid,
        in_specs=[
            row_spec,
            pl.BlockSpec((4, BR, 128), lambda i: (0, i, 0)),
            row_spec, row_spec, row_spec,
        ] + [smem] * 10,
        out_specs=[pl.BlockSpec((BR, 128), lambda i: (i, 0))] * 3,
        out_shape=[jax.ShapeDtypeStruct((ROWS2D + 50, 128), jnp.float32)] * 3,
    )(t2, g4, px, py, pz, *wb)


def _scatter_body(cx_hbm, cy_hbm, cz_hbm, b_hbm, bounds_hbm, out_hbm,
                  bounds_v, idx_v, c_v, acc):
    cid = lax.axis_index("c")
    sid = lax.axis_index("s")
    wid = sid * 2 + cid
    pltpu.sync_copy(bounds_hbm, bounds_v)
    bv = bounds_v[pl.ds(wid, 16)]
    r_lo = bv[0]
    r_hi = bv[1]
    start = (r_lo // 8) * 8
    nch = (r_hi - start + KCH - 1) // KCH
    wbase = wid * WIN

    def zero(i, _):
        acc[pl.ds(i * 16, 16)] = jnp.zeros((16,), jnp.float32)
        return 0

    lax.fori_loop(0, WIN * 4 // 16, zero, 0)

    lane = lax.iota(jnp.int32, 16)

    def chunk(k, _):
        off = start + k * KCH
        pltpu.sync_copy(b_hbm.at[pl.ds(off, KCH)], idx_v)
        pltpu.sync_copy(cx_hbm.at[pl.ds(off, KCH)], c_v.at[0])
        pltpu.sync_copy(cy_hbm.at[pl.ds(off, KCH)], c_v.at[1])
        pltpu.sync_copy(cz_hbm.at[pl.ds(off, KCH)], c_v.at[2])

        def grp(i, _):
            ids = idx_v[pl.ds(i * 16, 16)]
            rowpos = lane + (i * 16 + off)
            valid = (rowpos >= r_lo) & (rowpos < r_hi)
            local = jnp.where(valid, ids - wbase, 0)
            flat = local * 4
            for c in range(3):
                vals = c_v[c, pl.ds(i * 16, 16)]
                plsc.addupdate_scatter(acc, [flat + c], vals, mask=valid)
            return 0

        lax.fori_loop(0, KCH // 16, grp, 0)
        return 0

    lax.fori_loop(0, nch, chunk, 0)
    pltpu.sync_copy(acc, out_hbm.at[pl.ds(wid * WIN * 4, WIN * 4)])


_scatter_call = pl.kernel(
    _scatter_body,
    out_type=jax.ShapeDtypeStruct((Bp * 4,), jnp.float32),
    mesh=plsc.VectorSubcoreMesh(core_axis_name="c", subcore_axis_name="s"),
    compiler_params=_SC_PARAMS,
    scratch_types=[
        pltpu.VMEM((48,), jnp.int32),
        pltpu.VMEM((KCH,), jnp.int32),
        pltpu.VMEM((3, KCH), jnp.float32),
        pltpu.VMEM((WIN * 4,), jnp.float32),
    ],
)


def _norm_body(a_ref, o_ref):
    p = a_ref[...]
    sq = p * p
    r = lax.broadcasted_iota(jnp.int32, (128, 128), 0)
    c = lax.broadcasted_iota(jnp.int32, (128, 128), 1)
    M = ((r // 4) == (c // 4)).astype(jnp.float32)
    n2 = lax.dot_general(sq, M, (((1,), (0,)), ((), ())),
                         preferred_element_type=jnp.float32)
    n = jnp.sqrt(jnp.maximum(n2, 1e-24))
    o_ref[...] = p / n


def _norm_call(a):
    BRn = 400
    rows = (Bp * 4) // 128
    return pl.pallas_call(
        _norm_body,
        grid=(rows // BRn,),
        in_specs=[pl.BlockSpec((BRn, 128), lambda i: (i, 0))],
        out_specs=pl.BlockSpec((BRn, 128), lambda i: (i, 0)),
        out_shape=jax.ShapeDtypeStruct((rows, 128), jnp.float32),
    )(a)


def kernel(t, pos, poi_t, poi_pos, batch, W0, b0, W1, b1, W2, b2, W3, b3, W4, b4):
    f32 = jnp.float32
    tab = jnp.concatenate(
        [poi_t[:, None], poi_pos, jnp.zeros((B, 4), f32)], axis=1)
    tab = jnp.pad(tab, ((0, Bp - B), (0, 0)))

    batch_pp = jnp.concatenate(
        [batch, jnp.full((Npp - N,), Bp - 1, jnp.int32)])
    batch_p = batch_pp[:Np]

    g4 = _gather_call(tab, batch_p)

    t2 = jnp.pad(t, (0, Np - N)).reshape(ROWS2D, 128)
    px = jnp.pad(pos[:, 0], (0, Np - N)).reshape(ROWS2D, 128)
    py = jnp.pad(pos[:, 1], (0, Np - N)).reshape(ROWS2D, 128)
    pz = jnp.pad(pos[:, 2], (0, Np - N)).reshape(ROWS2D, 128)
    g4r = g4.reshape(4, ROWS2D, 128)

    cx, cy, cz = _mlp_call(t2, g4r, px, py, pz,
                           W0, b0, W1, b1, W2, b2, W3, b3, W4, b4)

    bounds = jnp.searchsorted(
        batch_p, jnp.arange(33, dtype=jnp.int32) * WIN).astype(jnp.int32)
    bounds = jnp.pad(bounds, (0, 15))

    acc = _scatter_call(cx.reshape(Npp), cy.reshape(Npp), cz.reshape(Npp),
                        batch_pp, bounds)

    o = _norm_call(acc.reshape((Bp * 4) // 128, 128))
    return o.reshape(Bp, 4)[:B, :3]


# speedup vs baseline: 6.4832x; 1.0067x over previous
import jax
import jax.numpy as jnp
from jax import lax
from jax.experimental import pallas as pl
from jax.experimental.pallas import tpu as pltpu
from jax.experimental.pallas import tpu_sc as plsc

N = 1_600_000
B = 100_000
Np = 1_638_400
Bp = 102_400
ROWS2D = Np // 128
NW = 32
RPT = Np // NW
KCH = 6400
NCH = RPT // KCH
WIN = Bp // NW
Npp = Np + KCH

_SC_PARAMS = pltpu.CompilerParams(use_tc_tiling_on_sc=False,
                                  needs_layout_passes=False)


def _gather_body(tab_hbm, batch_hbm, out_hbm, idx_v, rows_v, pln_v, sem, osem):
    cid = lax.axis_index("c")
    sid = lax.axis_index("s")
    wid = sid * 2 + cid
    lane = lax.iota(jnp.int32, 16)
    pend = []
    for k in range(NCH):
        base = wid * RPT + k * KCH
        buf = k % 2
        pltpu.sync_copy(batch_hbm.at[pl.ds(base, KCH)], idx_v)
        pltpu.async_copy(tab_hbm.at[idx_v], rows_v, sem).wait()
        if k >= 2:
            for d in pend[k - 2]:
                d.wait()

        def grp(i, _):
            r16 = lane + i * 16
            for c in range(4):
                v = plsc.load_gather(rows_v, [r16, jnp.full((16,), c, jnp.int32)])
                pln_v[buf, c, pl.ds(i * 16, 16)] = v
            return 0

        lax.fori_loop(0, KCH // 16, grp, 0)
        pend.append([
            pltpu.async_copy(pln_v.at[buf, j], out_hbm.at[j, pl.ds(base, KCH)],
                             osem)
            for j in range(4)
        ])
    for k in (NCH - 2, NCH - 1):
        for d in pend[k]:
            d.wait()


_gather_call = pl.kernel(
    _gather_body,
    out_type=jax.ShapeDtypeStruct((4, Np), jnp.float32),
    mesh=plsc.VectorSubcoreMesh(core_axis_name="c", subcore_axis_name="s"),
    compiler_params=_SC_PARAMS,
    scratch_types=[
        pltpu.VMEM((KCH,), jnp.int32),
        pltpu.VMEM((KCH, 8), jnp.float32),
        pltpu.VMEM((2, 4, KCH), jnp.float32),
        pltpu.SemaphoreType.DMA,
        pltpu.SemaphoreType.DMA,
    ],
)


def _mlp_body(t_ref, g_ref, px_ref, py_ref, pz_ref,
              W0, b0, W1, b1, W2, b2, W3, b3, W4, b4,
              cx_ref, cy_ref, cz_ref):
    tb = t_ref[...]
    gt, gx, gy, gz = g_ref[0], g_ref[1], g_ref[2], g_ref[3]
    px, py, pz = px_ref[...], py_ref[...], pz_ref[...]
    s = jnp.sign(tb - gt)
    dx = px - gx
    dy = py - gy
    dz = pz - gz
    r2 = dx * dx + dy * dy + dz * dz

    h = [s, r2]
    for W, b, fin, fout, relu in (
        (W0, b0, 2, 10, True),
        (W1, b1, 10, 20, True),
        (W2, b2, 20, 10, True),
        (W3, b3, 10, 5, True),
        (W4, b4, 5, 1, False),
    ):
        nxt = []
        for j in range(fout):
            acc = h[0] * W[j, 0] + b[j]
            for k in range(1, fin):
                acc = acc + h[k] * W[j, k]
            nxt.append(jnp.maximum(acc, 0.0) if relu else acc)
        h = nxt

    f = h[0] * lax.rsqrt(jnp.maximum(r2, 1e-24))
    cx_ref[...] = f * dx
    cy_ref[...] = f * dy
    cz_ref[...] = f * dz


def _mlp_call(t2, g4, px, py, pz, *wb):
    BR = 512
    grid = (ROWS2D // BR,)
    smem = pl.BlockSpec(memory_space=pltpu.MemorySpace.SMEM)
    row_spec = pl.BlockSpec((BR, 128), lambda i: (i, 0))
    return pl.pallas_call(
        _mlp_body,
        grid=grid,
        in_specs=[
            row_spec,
            pl.BlockSpec((4, BR, 128), lambda i: (0, i, 0)),
            row_spec, row_spec, row_spec,
        ] + [smem] * 10,
        out_specs=[pl.BlockSpec((BR, 128), lambda i: (i, 0))] * 3,
        out_shape=[jax.ShapeDtypeStruct((ROWS2D + 50, 128), jnp.float32)] * 3,
    )(t2, g4, px, py, pz, *wb)


def _scatter_body(cx_hbm, cy_hbm, cz_hbm, b_hbm, bounds_hbm, out_hbm,
                  bounds_v, idx_v, c_v, acc):
    cid = lax.axis_index("c")
    sid = lax.axis_index("s")
    wid = sid * 2 + cid
    pltpu.sync_copy(bounds_hbm, bounds_v)
    bv = bounds_v[pl.ds(wid, 16)]
    r_lo = bv[0]
    r_hi = bv[1]
    start = (r_lo // 8) * 8
    nch = (r_hi - start + KCH - 1) // KCH
    wbase = wid * WIN

    def zero(i, _):
        acc[pl.ds(i * 16, 16)] = jnp.zeros((16,), jnp.float32)
        return 0

    lax.fori_loop(0, WIN * 4 // 16, zero, 0)

    lane = lax.iota(jnp.int32, 16)

    def chunk(k, _):
        off = start + k * KCH
        pltpu.sync_copy(b_hbm.at[pl.ds(off, KCH)], idx_v)
        pltpu.sync_copy(cx_hbm.at[pl.ds(off, KCH)], c_v.at[0])
        pltpu.sync_copy(cy_hbm.at[pl.ds(off, KCH)], c_v.at[1])
        pltpu.sync_copy(cz_hbm.at[pl.ds(off, KCH)], c_v.at[2])

        def grp(i, _):
            ids = idx_v[pl.ds(i * 16, 16)]
            rowpos = lane + (i * 16 + off)
            valid = (rowpos >= r_lo) & (rowpos < r_hi)
            local = jnp.where(valid, ids - wbase, 0)
            flat = local * 4
            for c in range(3):
                vals = c_v[c, pl.ds(i * 16, 16)]
                plsc.addupdate_scatter(acc, [flat + c], vals, mask=valid)
            return 0

        lax.fori_loop(0, KCH // 16, grp, 0)
        return 0

    lax.fori_loop(0, nch, chunk, 0)
    pltpu.sync_copy(acc, out_hbm.at[pl.ds(wid * WIN * 4, WIN * 4)])


_scatter_call = pl.kernel(
    _scatter_body,
    out_type=jax.ShapeDtypeStruct((Bp * 4,), jnp.float32),
    mesh=plsc.VectorSubcoreMesh(core_axis_name="c", subcore_axis_name="s"),
    compiler_params=_SC_PARAMS,
    scratch_types=[
        pltpu.VMEM((48,), jnp.int32),
        pltpu.VMEM((KCH,), jnp.int32),
        pltpu.VMEM((3, KCH), jnp.float32),
        pltpu.VMEM((WIN * 4,), jnp.float32),
    ],
)


def _norm_body(a_ref, o_ref):
    p = a_ref[...]
    sq = p * p
    r = lax.broadcasted_iota(jnp.int32, (128, 128), 0)
    c = lax.broadcasted_iota(jnp.int32, (128, 128), 1)
    M = ((r // 4) == (c // 4)).astype(jnp.float32)
    n2 = lax.dot_general(sq, M, (((1,), (0,)), ((), ())),
                         preferred_element_type=jnp.float32)
    n = jnp.sqrt(jnp.maximum(n2, 1e-24))
    o_ref[...] = p / n


def _norm_call(a):
    BRn = 400
    rows = (Bp * 4) // 128
    return pl.pallas_call(
        _norm_body,
        grid=(rows // BRn,),
        in_specs=[pl.BlockSpec((BRn, 128), lambda i: (i, 0))],
        out_specs=pl.BlockSpec((BRn, 128), lambda i: (i, 0)),
        out_shape=jax.ShapeDtypeStruct((rows, 128), jnp.float32),
    )(a)


def kernel(t, pos, poi_t, poi_pos, batch, W0, b0, W1, b1, W2, b2, W3, b3, W4, b4):
    f32 = jnp.float32
    tab = jnp.concatenate(
        [poi_t[:, None], poi_pos, jnp.zeros((B, 4), f32)], axis=1)
    tab = jnp.pad(tab, ((0, Bp - B), (0, 0)))

    batch_pp = jnp.concatenate(
        [batch, jnp.full((Npp - N,), Bp - 1, jnp.int32)])
    batch_p = batch_pp[:Np]

    g4 = _gather_call(tab, batch_p)

    t2 = jnp.pad(t, (0, Np - N)).reshape(ROWS2D, 128)
    px = jnp.pad(pos[:, 0], (0, Np - N)).reshape(ROWS2D, 128)
    py = jnp.pad(pos[:, 1], (0, Np - N)).reshape(ROWS2D, 128)
    pz = jnp.pad(pos[:, 2], (0, Np - N)).reshape(ROWS2D, 128)
    g4r = g4.reshape(4, ROWS2D, 128)

    cx, cy, cz = _mlp_call(t2, g4r, px, py, pz,
                           W0, b0, W1, b1, W2, b2, W3, b3, W4, b4)

    bounds = jnp.searchsorted(
        batch_p, jnp.arange(33, dtype=jnp.int32) * WIN).astype(jnp.int32)
    bounds = jnp.pad(bounds, (0, 15))

    acc = _scatter_call(cx.reshape(Npp), cy.reshape(Npp), cz.reshape(Npp),
                        batch_pp, bounds)

    o = _norm_call(acc.reshape((Bp * 4) // 128, 128))
    return o.reshape(Bp, 4)[:B, :3]
